# baseline (speedup 1.0000x reference)
import functools

import jax
import jax.numpy as jnp
from jax import lax
from jax.experimental import pallas as pl
from jax.experimental.pallas import tpu as pltpu
from jax.experimental.pallas import tpu_sc as plsc

EMB_DIM = 64
BATCH = 16384

NUM_CORES = 2
NUM_SUBCORES = 16
LANES = 16
NW = NUM_CORES * NUM_SUBCORES
B_PER_W = BATCH // NW
CHUNK = 128
NCHUNK = B_PER_W // CHUNK
GROUPS = CHUNK // LANES

_mesh = plsc.VectorSubcoreMesh(
    core_axis_name="c", subcore_axis_name="s",
    num_cores=NUM_CORES, num_subcores=NUM_SUBCORES,
)


@functools.partial(
    pl.kernel,
    out_type=jax.ShapeDtypeStruct((BATCH,), jnp.float32),
    mesh=_mesh,
    scratch_types=[
        pltpu.VMEM((NCHUNK, CHUNK), jnp.int32),
        pltpu.VMEM((NCHUNK, CHUNK), jnp.int32),
        pltpu.VMEM((2, EMB_DIM, CHUNK), jnp.float32),
        pltpu.VMEM((2, EMB_DIM, CHUNK), jnp.float32),
        pltpu.VMEM((NCHUNK, CHUNK), jnp.float32),
        pltpu.VMEM((NCHUNK, CHUNK), jnp.float32),
        pltpu.VMEM((B_PER_W,), jnp.float32),
        pltpu.SemaphoreType.DMA,
        pltpu.SemaphoreType.DMA,
    ],
    compiler_params=pltpu.CompilerParams(
        needs_layout_passes=False, use_tc_tiling_on_sc=False),
)
def _ranker_sc(x1_hbm, x2_hbm, ut_hbm, ct_hbm, ubias_hbm, cbias_hbm,
               out_hbm, idx1_v, idx2_v, u_v, c_v, ub_v, cb_v, out_v,
               sem0, sem1):
    wid = lax.axis_index("s") * NUM_CORES + lax.axis_index("c")
    base = wid * B_PER_W
    sems = [sem0, sem1]

    for j in range(NCHUNK):
        pltpu.sync_copy(x1_hbm.at[pl.ds(base + j * CHUNK, CHUNK)], idx1_v.at[j])
        pltpu.sync_copy(x2_hbm.at[pl.ds(base + j * CHUNK, CHUNK)], idx2_v.at[j])

    def issue(j, buf, sem):
        idxu = idx1_v.at[j]
        idxc = idx2_v.at[j]

        @pl.loop(0, EMB_DIM)
        def _(d):
            pltpu.async_copy(ut_hbm.at[d].at[idxu], u_v.at[buf, d], sem)
            pltpu.async_copy(ct_hbm.at[d].at[idxc], c_v.at[buf, d], sem)

        pltpu.async_copy(ubias_hbm.at[idxu], ub_v.at[j], sem)
        pltpu.async_copy(cbias_hbm.at[idxc], cb_v.at[j], sem)

    def drain(j, buf, sem):
        pltpu.make_async_copy(
            ut_hbm.at[pl.ds(0, EMB_DIM), pl.ds(0, CHUNK)], u_v.at[buf], sem).wait()
        pltpu.make_async_copy(
            ct_hbm.at[pl.ds(0, EMB_DIM), pl.ds(0, CHUNK)], c_v.at[buf], sem).wait()
        pltpu.make_async_copy(ubias_hbm.at[pl.ds(0, CHUNK)], ub_v.at[j], sem).wait()
        pltpu.make_async_copy(cbias_hbm.at[pl.ds(0, CHUNK)], cb_v.at[j], sem).wait()

    def compute(j, buf):
        @pl.loop(0, GROUPS)
        def _(g):
            col = pl.ds(g * LANES, LANES)
            acc = u_v[buf, 0, col] * c_v[buf, 0, col]
            for d in range(1, EMB_DIM):
                acc = acc + u_v[buf, d, col] * c_v[buf, d, col]
            tot = acc + float(EMB_DIM) * (ub_v[j, col] + cb_v[j, col])
            out_v[pl.ds(j * CHUNK + g * LANES, LANES)] = 1.0 / (1.0 + jnp.exp(-tot))

    issue(0, 0, sems[0])
    for j in range(NCHUNK):
        if j + 1 < NCHUNK:
            issue(j + 1, (j + 1) % 2, sems[(j + 1) % 2])
        drain(j, j % 2, sems[j % 2])
        compute(j, j % 2)

    pltpu.sync_copy(out_v, out_hbm.at[pl.ds(base, B_PER_W)])


def kernel(x1, x2, uemb, cemb, user_bias, creator_bias):
    x1 = x1.astype(jnp.int32)
    x2 = x2.astype(jnp.int32)
    return _ranker_sc(x1, x2, uemb.T, cemb.T,
                      user_bias.reshape(-1), creator_bias.reshape(-1))


# speedup vs baseline: 3.4412x; 3.4412x over previous
import functools

import jax
import jax.numpy as jnp
from jax import lax
from jax.experimental import pallas as pl
from jax.experimental.pallas import tpu as pltpu
from jax.experimental.pallas import tpu_sc as plsc

EMB_DIM = 64
BATCH = 16384

NUM_CORES = 2
NUM_SUBCORES = 16
LANES = 16
NW = NUM_CORES * NUM_SUBCORES
B_PER_W = BATCH // NW
CHUNK = 128
NCHUNK = B_PER_W // CHUNK
GROUPS = CHUNK // LANES
VROW = 2 * EMB_DIM

_mesh = plsc.VectorSubcoreMesh(
    core_axis_name="c", subcore_axis_name="s",
    num_cores=NUM_CORES, num_subcores=NUM_SUBCORES,
)


@functools.partial(
    pl.kernel,
    out_type=jax.ShapeDtypeStruct((BATCH,), jnp.float32),
    mesh=_mesh,
    scratch_types=[
        pltpu.VMEM((NCHUNK, CHUNK), jnp.int32),
        pltpu.VMEM((NCHUNK, CHUNK), jnp.int32),
        pltpu.VMEM((NCHUNK, CHUNK), jnp.float32),
        pltpu.VMEM((NCHUNK, CHUNK), jnp.float32),
        pltpu.SemaphoreType.DMA,
    ],
    compiler_params=pltpu.CompilerParams(
        needs_layout_passes=False, use_tc_tiling_on_sc=False),
)
def _bias_sc(x1_hbm, x2_hbm, ubias_hbm, cbias_hbm, out_hbm,
             idx1_v, idx2_v, b1_v, b2_v, sem):
    wid = lax.axis_index("s") * NUM_CORES + lax.axis_index("c")
    base = wid * B_PER_W

    for j in range(NCHUNK):
        pltpu.sync_copy(x1_hbm.at[pl.ds(base + j * CHUNK, CHUNK)], idx1_v.at[j])
        pltpu.sync_copy(x2_hbm.at[pl.ds(base + j * CHUNK, CHUNK)], idx2_v.at[j])

    copies = []
    for j in range(NCHUNK):
        copies.append(pltpu.async_copy(ubias_hbm.at[idx1_v.at[j]], b1_v.at[j], sem))
        copies.append(pltpu.async_copy(cbias_hbm.at[idx2_v.at[j]], b2_v.at[j], sem))
    for cp in copies:
        cp.wait()

    for j in range(NCHUNK):
        @pl.loop(0, GROUPS)
        def _(g):
            col = pl.ds(g * LANES, LANES)
            b1_v[j, col] = b1_v[j, col] + b2_v[j, col]

        pltpu.sync_copy(b1_v.at[j], out_hbm.at[pl.ds(base + j * CHUNK, CHUNK)])


@functools.partial(
    pl.kernel,
    out_type=jax.ShapeDtypeStruct((BATCH,), jnp.float32),
    mesh=_mesh,
    scratch_types=[
        pltpu.VMEM((NCHUNK, CHUNK), jnp.int32),
        pltpu.VMEM((NCHUNK, CHUNK), jnp.int32),
        pltpu.VMEM((NCHUNK, CHUNK), jnp.int32),
        pltpu.VMEM((NCHUNK, CHUNK), jnp.int32),
        pltpu.VMEM((2, CHUNK, VROW), jnp.float32),
        pltpu.VMEM((2, CHUNK, VROW), jnp.float32),
        pltpu.VMEM((NCHUNK, CHUNK), jnp.float32),
        pltpu.VMEM((B_PER_W,), jnp.float32),
        pltpu.VMEM((LANES, LANES + 1), jnp.float32),
        pltpu.SemaphoreType.DMA,
        pltpu.SemaphoreType.DMA,
    ],
    compiler_params=pltpu.CompilerParams(
        needs_layout_passes=False, use_tc_tiling_on_sc=True),
)
def _ranker_sc(x1_hbm, x2_hbm, up_hbm, cp_hbm, bsum_hbm,
               out_hbm, idx1_v, idx2_v, idx1p_v, idx2p_v, u_v, c_v, bs_v,
               out_v, pad_v, sem0, sem1):
    wid = lax.axis_index("s") * NUM_CORES + lax.axis_index("c")
    base = wid * B_PER_W
    sems = [sem0, sem1]

    for j in range(NCHUNK):
        pltpu.sync_copy(x1_hbm.at[pl.ds(base + j * CHUNK, CHUNK)], idx1_v.at[j])
        pltpu.sync_copy(x2_hbm.at[pl.ds(base + j * CHUNK, CHUNK)], idx2_v.at[j])
        pltpu.sync_copy(bsum_hbm.at[pl.ds(base + j * CHUNK, CHUNK)], bs_v.at[j])

        @pl.loop(0, GROUPS)
        def _(g):
            col = pl.ds(g * LANES, LANES)
            idx1p_v[j, col] = lax.shift_right_logical(idx1_v[j, col], 1)
            idx2p_v[j, col] = lax.shift_right_logical(idx2_v[j, col], 1)

    def issue(j, buf, sem):
        pltpu.async_copy(up_hbm.at[idx1p_v.at[j]], u_v.at[buf], sem)
        pltpu.async_copy(cp_hbm.at[idx2p_v.at[j]], c_v.at[buf], sem)

    def drain(j, buf, sem):
        pltpu.make_async_copy(
            up_hbm.at[pl.ds(0, CHUNK), pl.ds(0, VROW)], u_v.at[buf], sem).wait()
        pltpu.make_async_copy(
            cp_hbm.at[pl.ds(0, CHUNK), pl.ds(0, VROW)], c_v.at[buf], sem).wait()

    iota16 = lax.iota(jnp.int32, LANES)

    def compute(j, buf):
        @pl.loop(0, GROUPS)
        def _(g):
            rbase = g * LANES
            par1 = (idx1_v[j, pl.ds(rbase, LANES)] & 1) * EMB_DIM
            par2 = (idx2_v[j, pl.ds(rbase, LANES)] & 1) * EMB_DIM
            for i in range(LANES):
                r = rbase + i
                uoff = par1[i]
                coff = par2[i]
                acc = (u_v[buf, r, pl.ds(uoff, LANES)]
                       * c_v[buf, r, pl.ds(coff, LANES)])
                for k in range(1, EMB_DIM // LANES):
                    acc = acc + (u_v[buf, r, pl.ds(uoff + k * LANES, LANES)]
                                 * c_v[buf, r, pl.ds(coff + k * LANES, LANES)])
                plsc.store_scatter(
                    pad_v, [iota16, jnp.full((LANES,), i, jnp.int32)], acc)
            dots = pad_v[0, pl.ds(0, LANES)]
            for l in range(1, LANES):
                dots = dots + pad_v[l, pl.ds(0, LANES)]
            col = pl.ds(rbase, LANES)
            tot = dots + float(EMB_DIM) * bs_v[j, col]
            out_v[pl.ds(j * CHUNK + rbase, LANES)] = 1.0 / (1.0 + jnp.exp(-tot))

    issue(0, 0, sems[0])
    for j in range(NCHUNK):
        if j + 1 < NCHUNK:
            issue(j + 1, (j + 1) % 2, sems[(j + 1) % 2])
        drain(j, j % 2, sems[j % 2])
        compute(j, j % 2)

    pltpu.sync_copy(out_v, out_hbm.at[pl.ds(base, B_PER_W)])


N_U = 1000000
N_C = 100000
FULL_RB_U = N_U // CHUNK
FULL_RB_C = N_C // CHUNK
TAIL_U = N_U - FULL_RB_U * CHUNK
TAIL_C = N_C - FULL_RB_C * CHUNK
PER_U, EXTRA_U = FULL_RB_U // NW, FULL_RB_U % NW
PER_C, EXTRA_C = FULL_RB_C // NW, FULL_RB_C % NW


@functools.partial(
    pl.kernel,
    out_type=[
        jax.ShapeDtypeStruct((N_U // 2, VROW), jnp.float32),
        jax.ShapeDtypeStruct((N_C // 2, VROW), jnp.float32),
    ],
    mesh=_mesh,
    scratch_types=[
        pltpu.VMEM((2, EMB_DIM, CHUNK), jnp.float32),
        pltpu.VMEM((2, EMB_DIM, VROW), jnp.float32),
        pltpu.SemaphoreType.DMA,
        pltpu.SemaphoreType.DMA,
        pltpu.SemaphoreType.DMA,
    ],
    compiler_params=pltpu.CompilerParams(
        needs_layout_passes=False, use_tc_tiling_on_sc=True),
)
def _transpose_sc(ut_hbm, ct_hbm, tail_u_hbm, tail_c_hbm, up_hbm, cp_hbm,
                  tb, tbT, semi0, semi1, semo):
    wid = lax.axis_index("s") * NUM_CORES + lax.axis_index("c")
    iota16 = lax.iota(jnp.int32, LANES)

    def transpose_buf(buf, nrows, base_e=0):
        @pl.loop(0, nrows)
        def _(tr):
            e0 = iota16 * 0 + base_e + 2 * tr
            for k in range(EMB_DIM // LANES):
                fidx = k * LANES + iota16
                ge = plsc.load_gather(tb.at[buf], [fidx, e0])
                go = plsc.load_gather(tb.at[buf], [fidx, e0 + 1])
                tbT[buf, tr, pl.ds(k * LANES, LANES)] = ge
                tbT[buf, tr, pl.ds(EMB_DIM + k * LANES, LANES)] = go

    def process(src, dst, per, extra, tail_src, tail, tail_dst_off):
        lo = wid * per + jnp.minimum(wid, extra)
        n = per + (wid < extra).astype(jnp.int32)
        sems = (semi0, semi1)

        def issue_in(r, buf):
            off = pl.multiple_of(r * CHUNK, CHUNK)
            for td in range(EMB_DIM // 8):
                pltpu.async_copy(
                    src.at[pl.ds(8 * td, 8), pl.ds(off, CHUNK)],
                    tb.at[buf, pl.ds(8 * td, 8), pl.ds(0, CHUNK)], sems[buf])

        def drain_in(buf):
            pltpu.make_async_copy(
                src.at[pl.ds(0, EMB_DIM), pl.ds(0, CHUNK)],
                tb.at[buf, pl.ds(0, EMB_DIM), pl.ds(0, CHUNK)], sems[buf]).wait()

        def issue_out(r, buf):
            off = pl.multiple_of(r * (CHUNK // 2), CHUNK // 2)
            pltpu.async_copy(
                tbT.at[buf], dst.at[pl.ds(off, CHUNK // 2)], semo)

        def drain_out():
            pltpu.make_async_copy(
                dst.at[pl.ds(0, CHUNK // 2), pl.ds(0, VROW)], tbT.at[0], semo).wait()

        m = (n + 1) // 2

        @pl.loop(0, m)
        def _(i):
            r0 = lo + 2 * i
            v1 = 2 * i + 1 < n
            issue_in(r0, 0)

            @pl.when(v1)
            def _():
                issue_in(r0 + 1, 1)

            @pl.when(i > 0)
            def _():
                drain_out()
                drain_out()

            drain_in(0)
            transpose_buf(0, CHUNK // 2)
            issue_out(r0, 0)

            @pl.when(v1)
            def _():
                drain_in(1)
                transpose_buf(1, CHUNK // 2)
                issue_out(r0 + 1, 1)

        dn = n - 2 * (m - 1)

        @pl.when(dn >= 1)
        def _():
            drain_out()

        @pl.when(dn >= 2)
        def _():
            drain_out()

        nt = tail // 2

        @pl.when(wid == NW - 1)
        def _():
            pltpu.sync_copy(tail_src, tbT.at[0, pl.ds(0, nt), pl.ds(0, VROW)])
            pltpu.sync_copy(tbT.at[0, pl.ds(0, nt), pl.ds(0, VROW)],
                            dst.at[pl.ds(tail_dst_off, nt)])

    process(ut_hbm, up_hbm, PER_U, EXTRA_U, tail_u_hbm, TAIL_U,
            FULL_RB_U * (CHUNK // 2))
    process(ct_hbm, cp_hbm, PER_C, EXTRA_C, tail_c_hbm, TAIL_C,
            FULL_RB_C * (CHUNK // 2))


def kernel(x1, x2, uemb, cemb, user_bias, creator_bias):
    x1 = x1.astype(jnp.int32)
    x2 = x2.astype(jnp.int32)
    bsum = _bias_sc(x1, x2, user_bias.reshape(-1), creator_bias.reshape(-1))
    tail_u = uemb[FULL_RB_U * CHUNK:].reshape(TAIL_U // 2, VROW)
    tail_c = cemb[FULL_RB_C * CHUNK:].reshape(TAIL_C // 2, VROW)
    up, cp = _transpose_sc(uemb.T, cemb.T, tail_u, tail_c)
    return _ranker_sc(x1, x2, up, cp, bsum)


# speedup vs baseline: 11.7816x; 3.4237x over previous
import functools

import jax
import jax.numpy as jnp
from jax import lax
from jax.experimental import pallas as pl
from jax.experimental.pallas import tpu as pltpu
from jax.experimental.pallas import tpu_sc as plsc

EMB_DIM = 64
BATCH = 16384

NUM_CORES = 2
NUM_SUBCORES = 16
LANES = 16
NW = NUM_CORES * NUM_SUBCORES
B_PER_W = BATCH // NW
CHUNK = 128
NCHUNK = B_PER_W // CHUNK
GROUPS = CHUNK // LANES
VROW = 2 * EMB_DIM

_mesh = plsc.VectorSubcoreMesh(
    core_axis_name="c", subcore_axis_name="s",
    num_cores=NUM_CORES, num_subcores=NUM_SUBCORES,
)


@functools.partial(
    pl.kernel,
    out_type=jax.ShapeDtypeStruct((BATCH,), jnp.float32),
    mesh=_mesh,
    scratch_types=[
        pltpu.VMEM((NCHUNK, CHUNK), jnp.int32),
        pltpu.VMEM((NCHUNK, CHUNK), jnp.int32),
        pltpu.VMEM((NCHUNK, CHUNK), jnp.float32),
        pltpu.VMEM((NCHUNK, CHUNK), jnp.float32),
        pltpu.SemaphoreType.DMA,
    ],
    compiler_params=pltpu.CompilerParams(
        needs_layout_passes=False, use_tc_tiling_on_sc=False),
)
def _bias_sc(x1_hbm, x2_hbm, ubias_hbm, cbias_hbm, out_hbm,
             idx1_v, idx2_v, b1_v, b2_v, sem):
    wid = lax.axis_index("s") * NUM_CORES + lax.axis_index("c")
    base = wid * B_PER_W

    for j in range(NCHUNK):
        pltpu.sync_copy(x1_hbm.at[pl.ds(base + j * CHUNK, CHUNK)], idx1_v.at[j])
        pltpu.sync_copy(x2_hbm.at[pl.ds(base + j * CHUNK, CHUNK)], idx2_v.at[j])

    copies = []
    for j in range(NCHUNK):
        copies.append(pltpu.async_copy(ubias_hbm.at[idx1_v.at[j]], b1_v.at[j], sem))
        copies.append(pltpu.async_copy(cbias_hbm.at[idx2_v.at[j]], b2_v.at[j], sem))
    for cp in copies:
        cp.wait()

    for j in range(NCHUNK):
        @pl.loop(0, GROUPS)
        def _(g):
            col = pl.ds(g * LANES, LANES)
            b1_v[j, col] = b1_v[j, col] + b2_v[j, col]

        pltpu.sync_copy(b1_v.at[j], out_hbm.at[pl.ds(base + j * CHUNK, CHUNK)])


@functools.partial(
    pl.kernel,
    out_type=jax.ShapeDtypeStruct((BATCH,), jnp.float32),
    mesh=_mesh,
    scratch_types=[
        pltpu.VMEM((NCHUNK, CHUNK), jnp.int32),
        pltpu.VMEM((NCHUNK, CHUNK), jnp.int32),
        pltpu.VMEM((NCHUNK, CHUNK), jnp.int32),
        pltpu.VMEM((NCHUNK, CHUNK), jnp.int32),
        pltpu.VMEM((2, CHUNK, VROW), jnp.float32),
        pltpu.VMEM((2, CHUNK, VROW), jnp.float32),
        pltpu.VMEM((NCHUNK, CHUNK), jnp.float32),
        pltpu.VMEM((B_PER_W,), jnp.float32),
        pltpu.VMEM((LANES, LANES + 1), jnp.float32),
        pltpu.SemaphoreType.DMA,
        pltpu.SemaphoreType.DMA,
    ],
    compiler_params=pltpu.CompilerParams(
        needs_layout_passes=False, use_tc_tiling_on_sc=True),
)
def _ranker_sc(x1_hbm, x2_hbm, up_hbm, cp_hbm, bsum_hbm,
               out_hbm, idx1_v, idx2_v, idx1p_v, idx2p_v, u_v, c_v, bs_v,
               out_v, pad_v, sem0, sem1):
    wid = lax.axis_index("s") * NUM_CORES + lax.axis_index("c")
    base = wid * B_PER_W
    sems = [sem0, sem1]

    for j in range(NCHUNK):
        pltpu.sync_copy(x1_hbm.at[pl.ds(base + j * CHUNK, CHUNK)], idx1_v.at[j])
        pltpu.sync_copy(x2_hbm.at[pl.ds(base + j * CHUNK, CHUNK)], idx2_v.at[j])
        pltpu.sync_copy(bsum_hbm.at[pl.ds(base + j * CHUNK, CHUNK)], bs_v.at[j])

        @pl.loop(0, GROUPS)
        def _(g):
            col = pl.ds(g * LANES, LANES)
            v1 = idx1_v[j, col]
            v2 = idx2_v[j, col]
            idx1p_v[j, col] = lax.shift_left(
                lax.shift_right_logical(v1, 12), 11) + (v1 & (PACK - 1))
            idx2p_v[j, col] = lax.shift_left(
                lax.shift_right_logical(v2, 12), 11) + (v2 & (PACK - 1))

    def issue(j, buf, sem):
        pltpu.async_copy(up_hbm.at[idx1p_v.at[j]], u_v.at[buf], sem)
        pltpu.async_copy(cp_hbm.at[idx2p_v.at[j]], c_v.at[buf], sem)

    def drain(j, buf, sem):
        pltpu.make_async_copy(
            up_hbm.at[pl.ds(0, CHUNK), pl.ds(0, VROW)], u_v.at[buf], sem).wait()
        pltpu.make_async_copy(
            cp_hbm.at[pl.ds(0, CHUNK), pl.ds(0, VROW)], c_v.at[buf], sem).wait()

    iota16 = lax.iota(jnp.int32, LANES)

    def compute(j, buf):
        @pl.loop(0, GROUPS)
        def _(g):
            rbase = g * LANES
            par1 = (lax.shift_right_logical(
                idx1_v[j, pl.ds(rbase, LANES)], 11) & 1) * EMB_DIM
            par2 = (lax.shift_right_logical(
                idx2_v[j, pl.ds(rbase, LANES)], 11) & 1) * EMB_DIM
            for i in range(LANES):
                r = rbase + i
                uoff = par1[i]
                coff = par2[i]
                acc = (u_v[buf, r, pl.ds(uoff, LANES)]
                       * c_v[buf, r, pl.ds(coff, LANES)])
                for k in range(1, EMB_DIM // LANES):
                    acc = acc + (u_v[buf, r, pl.ds(uoff + k * LANES, LANES)]
                                 * c_v[buf, r, pl.ds(coff + k * LANES, LANES)])
                plsc.store_scatter(
                    pad_v, [iota16, jnp.full((LANES,), i, jnp.int32)], acc)
            dots = pad_v[0, pl.ds(0, LANES)]
            for l in range(1, LANES):
                dots = dots + pad_v[l, pl.ds(0, LANES)]
            col = pl.ds(rbase, LANES)
            tot = dots + float(EMB_DIM) * bs_v[j, col]
            out_v[pl.ds(j * CHUNK + rbase, LANES)] = 1.0 / (1.0 + jnp.exp(-tot))

    issue(0, 0, sems[0])
    for j in range(NCHUNK):
        if j + 1 < NCHUNK:
            issue(j + 1, (j + 1) % 2, sems[(j + 1) % 2])
        drain(j, j % 2, sems[j % 2])
        compute(j, j % 2)

    pltpu.sync_copy(out_v, out_hbm.at[pl.ds(base, B_PER_W)])


N_U = 1000000
N_C = 100000
PACK = 2048
BLKL = 2 * PACK
NBLK_U = (N_U + BLKL - 1) // BLKL
NBLK_C = (N_C + BLKL - 1) // BLKL


def _repack_body(t_ref, o_ref):
    a = t_ref[:, :PACK]
    b = t_ref[:, PACK:]
    o_ref[...] = jnp.concatenate([a.T, b.T], axis=1)


def _repack_tc(table_t, nblk):
    return pl.pallas_call(
        _repack_body,
        grid=(nblk,),
        in_specs=[pl.BlockSpec((EMB_DIM, BLKL), lambda j: (0, j))],
        out_specs=pl.BlockSpec((PACK, VROW), lambda j: (j, 0)),
        out_shape=jax.ShapeDtypeStruct((nblk * PACK, VROW), jnp.float32),
    )(table_t)


def kernel(x1, x2, uemb, cemb, user_bias, creator_bias):
    x1 = x1.astype(jnp.int32)
    x2 = x2.astype(jnp.int32)
    bsum = _bias_sc(x1, x2, user_bias.reshape(-1), creator_bias.reshape(-1))
    up = _repack_tc(uemb.T, NBLK_U)
    cp = _repack_tc(cemb.T, NBLK_C)
    return _ranker_sc(x1, x2, up, cp, bsum)


# speedup vs baseline: 14.1034x; 1.1971x over previous
import functools

import jax
import jax.numpy as jnp
from jax import lax
from jax.experimental import pallas as pl
from jax.experimental.pallas import tpu as pltpu
from jax.experimental.pallas import tpu_sc as plsc

EMB_DIM = 64
BATCH = 16384

NUM_CORES = 2
NUM_SUBCORES = 16
LANES = 16
NW = NUM_CORES * NUM_SUBCORES
B_PER_W = BATCH // NW
CHUNK = 128
NCHUNK = B_PER_W // CHUNK
GROUPS = CHUNK // LANES
VROW = 2 * EMB_DIM

_mesh = plsc.VectorSubcoreMesh(
    core_axis_name="c", subcore_axis_name="s",
    num_cores=NUM_CORES, num_subcores=NUM_SUBCORES,
)


@functools.partial(
    pl.kernel,
    out_type=jax.ShapeDtypeStruct((BATCH,), jnp.float32),
    mesh=_mesh,
    scratch_types=[
        pltpu.VMEM((NCHUNK, CHUNK), jnp.int32),
        pltpu.VMEM((NCHUNK, CHUNK), jnp.int32),
        pltpu.VMEM((NCHUNK, CHUNK), jnp.float32),
        pltpu.VMEM((NCHUNK, CHUNK), jnp.float32),
        pltpu.SemaphoreType.DMA,
    ],
    compiler_params=pltpu.CompilerParams(
        needs_layout_passes=False, use_tc_tiling_on_sc=False),
)
def _bias_sc(x1_hbm, x2_hbm, ubias_hbm, cbias_hbm, out_hbm,
             idx1_v, idx2_v, b1_v, b2_v, sem):
    wid = lax.axis_index("s") * NUM_CORES + lax.axis_index("c")
    base = wid * B_PER_W

    for j in range(NCHUNK):
        pltpu.sync_copy(x1_hbm.at[pl.ds(base + j * CHUNK, CHUNK)], idx1_v.at[j])
        pltpu.sync_copy(x2_hbm.at[pl.ds(base + j * CHUNK, CHUNK)], idx2_v.at[j])

    copies = []
    for j in range(NCHUNK):
        copies.append(pltpu.async_copy(ubias_hbm.at[idx1_v.at[j]], b1_v.at[j], sem))
        copies.append(pltpu.async_copy(cbias_hbm.at[idx2_v.at[j]], b2_v.at[j], sem))
    for cp in copies:
        cp.wait()

    for j in range(NCHUNK):
        @pl.loop(0, GROUPS)
        def _(g):
            col = pl.ds(g * LANES, LANES)
            b1_v[j, col] = b1_v[j, col] + b2_v[j, col]

        pltpu.sync_copy(b1_v.at[j], out_hbm.at[pl.ds(base + j * CHUNK, CHUNK)])


@functools.partial(
    pl.kernel,
    out_type=jax.ShapeDtypeStruct((BATCH,), jnp.float32),
    mesh=_mesh,
    scratch_types=[
        pltpu.VMEM((NCHUNK, CHUNK), jnp.int32),
        pltpu.VMEM((NCHUNK, CHUNK), jnp.int32),
        pltpu.VMEM((NCHUNK, CHUNK), jnp.int32),
        pltpu.VMEM((NCHUNK, CHUNK), jnp.int32),
        pltpu.VMEM((2, CHUNK, VROW), jnp.float32),
        pltpu.VMEM((2, CHUNK, VROW), jnp.float32),
        pltpu.VMEM((NCHUNK, CHUNK), jnp.float32),
        pltpu.VMEM((B_PER_W,), jnp.float32),
        pltpu.VMEM((LANES, LANES + 1), jnp.float32),
        pltpu.SemaphoreType.DMA,
        pltpu.SemaphoreType.DMA,
    ],
    compiler_params=pltpu.CompilerParams(
        needs_layout_passes=False, use_tc_tiling_on_sc=True),
)
def _ranker_sc(x1_hbm, x2_hbm, up_hbm, cp_hbm, bsum_hbm,
               out_hbm, idx1_v, idx2_v, idx1p_v, idx2p_v, u_v, c_v, bs_v,
               out_v, pad_v, sem0, sem1):
    wid = lax.axis_index("s") * NUM_CORES + lax.axis_index("c")
    base = wid * B_PER_W
    sems = [sem0, sem1]

    for j in range(NCHUNK):
        pltpu.sync_copy(x1_hbm.at[pl.ds(base + j * CHUNK, CHUNK)], idx1_v.at[j])
        pltpu.sync_copy(x2_hbm.at[pl.ds(base + j * CHUNK, CHUNK)], idx2_v.at[j])
        pltpu.sync_copy(bsum_hbm.at[pl.ds(base + j * CHUNK, CHUNK)], bs_v.at[j])

        @pl.loop(0, GROUPS)
        def _(g):
            col = pl.ds(g * LANES, LANES)
            v1 = idx1_v[j, col]
            v2 = idx2_v[j, col]
            idx1p_v[j, col] = lax.shift_left(
                lax.shift_right_logical(v1, LOG2_BLKL), LOG2_PACK) + (
                    v1 & (PACK - 1))
            idx2p_v[j, col] = lax.shift_left(
                lax.shift_right_logical(v2, LOG2_BLKL), LOG2_PACK) + (
                    v2 & (PACK - 1))

    def issue(j, buf, sem):
        pltpu.async_copy(up_hbm.at[idx1p_v.at[j]], u_v.at[buf], sem)
        pltpu.async_copy(cp_hbm.at[idx2p_v.at[j]], c_v.at[buf], sem)

    def drain(j, buf, sem):
        pltpu.make_async_copy(
            up_hbm.at[pl.ds(0, CHUNK), pl.ds(0, VROW)], u_v.at[buf], sem).wait()
        pltpu.make_async_copy(
            cp_hbm.at[pl.ds(0, CHUNK), pl.ds(0, VROW)], c_v.at[buf], sem).wait()

    iota16 = lax.iota(jnp.int32, LANES)

    def compute(j, buf):
        @pl.loop(0, GROUPS)
        def _(g):
            rbase = g * LANES
            par1 = (lax.shift_right_logical(
                idx1_v[j, pl.ds(rbase, LANES)], LOG2_PACK) & 1) * EMB_DIM
            par2 = (lax.shift_right_logical(
                idx2_v[j, pl.ds(rbase, LANES)], LOG2_PACK) & 1) * EMB_DIM
            for i in range(LANES):
                r = rbase + i
                uoff = par1[i]
                coff = par2[i]
                acc = (u_v[buf, r, pl.ds(uoff, LANES)]
                       * c_v[buf, r, pl.ds(coff, LANES)])
                for k in range(1, EMB_DIM // LANES):
                    acc = acc + (u_v[buf, r, pl.ds(uoff + k * LANES, LANES)]
                                 * c_v[buf, r, pl.ds(coff + k * LANES, LANES)])
                plsc.store_scatter(
                    pad_v, [iota16, jnp.full((LANES,), i, jnp.int32)], acc)
            dots = pad_v[0, pl.ds(0, LANES)]
            for l in range(1, LANES):
                dots = dots + pad_v[l, pl.ds(0, LANES)]
            col = pl.ds(rbase, LANES)
            tot = dots + float(EMB_DIM) * bs_v[j, col]
            out_v[pl.ds(j * CHUNK + rbase, LANES)] = 1.0 / (1.0 + jnp.exp(-tot))

    issue(0, 0, sems[0])
    for j in range(NCHUNK):
        if j + 1 < NCHUNK:
            issue(j + 1, (j + 1) % 2, sems[(j + 1) % 2])
        drain(j, j % 2, sems[j % 2])
        compute(j, j % 2)

    pltpu.sync_copy(out_v, out_hbm.at[pl.ds(base, B_PER_W)])


N_U = 1000000
N_C = 100000
PACK = 4096
LOG2_PACK = 12
BLKL = 2 * PACK
LOG2_BLKL = LOG2_PACK + 1
NBLK_U = (N_U + BLKL - 1) // BLKL
NBLK_C = (N_C + BLKL - 1) // BLKL


def _repack_body(t_ref, o_ref):
    a = t_ref[:, :PACK]
    b = t_ref[:, PACK:]
    o_ref[...] = jnp.concatenate([a.T, b.T], axis=1)


def _repack_tc(table_t, nblk):
    return pl.pallas_call(
        _repack_body,
        grid=(nblk,),
        in_specs=[pl.BlockSpec((EMB_DIM, BLKL), lambda j: (0, j))],
        out_specs=pl.BlockSpec((PACK, VROW), lambda j: (j, 0)),
        out_shape=jax.ShapeDtypeStruct((nblk * PACK, VROW), jnp.float32),
        compiler_params=pltpu.CompilerParams(
            dimension_semantics=("parallel",)),
    )(table_t)


def kernel(x1, x2, uemb, cemb, user_bias, creator_bias):
    x1 = x1.astype(jnp.int32)
    x2 = x2.astype(jnp.int32)
    bsum = _bias_sc(x1, x2, user_bias.reshape(-1), creator_bias.reshape(-1))
    up = _repack_tc(uemb.T, NBLK_U)
    cp = _repack_tc(cemb.T, NBLK_C)
    return _ranker_sc(x1, x2, up, cp, bsum)


# speedup vs baseline: 15.4729x; 1.0971x over previous
import functools

import jax
import jax.numpy as jnp
from jax import lax
from jax.experimental import pallas as pl
from jax.experimental.pallas import tpu as pltpu
from jax.experimental.pallas import tpu_sc as plsc

EMB_DIM = 64
BATCH = 16384

NUM_CORES = 2
NUM_SUBCORES = 16
LANES = 16
NW = NUM_CORES * NUM_SUBCORES
B_PER_W = BATCH // NW
CHUNK = 128
NCHUNK = B_PER_W // CHUNK
GROUPS = CHUNK // LANES
VROW = 2 * EMB_DIM

_mesh = plsc.VectorSubcoreMesh(
    core_axis_name="c", subcore_axis_name="s",
    num_cores=NUM_CORES, num_subcores=NUM_SUBCORES,
)


@functools.partial(
    pl.kernel,
    out_type=jax.ShapeDtypeStruct((BATCH,), jnp.float32),
    mesh=_mesh,
    scratch_types=[
        pltpu.VMEM((NCHUNK, CHUNK), jnp.int32),
        pltpu.VMEM((NCHUNK, CHUNK), jnp.int32),
        pltpu.VMEM((NCHUNK, CHUNK), jnp.float32),
        pltpu.VMEM((NCHUNK, CHUNK), jnp.float32),
        pltpu.SemaphoreType.DMA,
    ],
    compiler_params=pltpu.CompilerParams(
        needs_layout_passes=False, use_tc_tiling_on_sc=False),
)
def _bias_sc(x1_hbm, x2_hbm, ubias_hbm, cbias_hbm, out_hbm,
             idx1_v, idx2_v, b1_v, b2_v, sem):
    wid = lax.axis_index("s") * NUM_CORES + lax.axis_index("c")
    base = wid * B_PER_W

    for j in range(NCHUNK):
        pltpu.sync_copy(x1_hbm.at[pl.ds(base + j * CHUNK, CHUNK)], idx1_v.at[j])
        pltpu.sync_copy(x2_hbm.at[pl.ds(base + j * CHUNK, CHUNK)], idx2_v.at[j])

    copies = []
    for j in range(NCHUNK):
        copies.append(pltpu.async_copy(ubias_hbm.at[idx1_v.at[j]], b1_v.at[j], sem))
        copies.append(pltpu.async_copy(cbias_hbm.at[idx2_v.at[j]], b2_v.at[j], sem))
    for cp in copies:
        cp.wait()

    for j in range(NCHUNK):
        @pl.loop(0, GROUPS)
        def _(g):
            col = pl.ds(g * LANES, LANES)
            b1_v[j, col] = b1_v[j, col] + b2_v[j, col]

        pltpu.sync_copy(b1_v.at[j], out_hbm.at[pl.ds(base + j * CHUNK, CHUNK)])


@functools.partial(
    pl.kernel,
    out_type=jax.ShapeDtypeStruct((BATCH,), jnp.float32),
    mesh=_mesh,
    scratch_types=[
        pltpu.VMEM((NCHUNK, CHUNK), jnp.int32),
        pltpu.VMEM((NCHUNK, CHUNK), jnp.int32),
        pltpu.VMEM((NCHUNK, CHUNK), jnp.int32),
        pltpu.VMEM((NCHUNK, CHUNK), jnp.int32),
        pltpu.VMEM((2, CHUNK, VROW), jnp.float32),
        pltpu.VMEM((2, CHUNK, VROW), jnp.float32),
        pltpu.VMEM((NCHUNK, CHUNK), jnp.float32),
        pltpu.VMEM((B_PER_W,), jnp.float32),
        pltpu.VMEM((LANES, LANES + 1), jnp.float32),
        pltpu.SemaphoreType.DMA,
        pltpu.SemaphoreType.DMA,
    ],
    compiler_params=pltpu.CompilerParams(
        needs_layout_passes=False, use_tc_tiling_on_sc=True),
)
def _ranker_sc(x1_hbm, x2_hbm, up_hbm, cp_hbm, bsum_hbm,
               out_hbm, idx1_v, idx2_v, idx1p_v, idx2p_v, u_v, c_v, bs_v,
               out_v, pad_v, sem0, sem1):
    wid = lax.axis_index("s") * NUM_CORES + lax.axis_index("c")
    base = wid * B_PER_W
    sems = [sem0, sem1]

    for j in range(NCHUNK):
        pltpu.sync_copy(x1_hbm.at[pl.ds(base + j * CHUNK, CHUNK)], idx1_v.at[j])
        pltpu.sync_copy(x2_hbm.at[pl.ds(base + j * CHUNK, CHUNK)], idx2_v.at[j])
        pltpu.sync_copy(bsum_hbm.at[pl.ds(base + j * CHUNK, CHUNK)], bs_v.at[j])

        @pl.loop(0, GROUPS)
        def _(g):
            col = pl.ds(g * LANES, LANES)
            v1 = idx1_v[j, col]
            v2 = idx2_v[j, col]
            idx1p_v[j, col] = lax.shift_left(
                lax.shift_right_logical(v1, LOG2_BLKL), LOG2_PACK) + (
                    v1 & (PACK - 1))
            idx2p_v[j, col] = lax.shift_left(
                lax.shift_right_logical(v2, LOG2_BLKL), LOG2_PACK) + (
                    v2 & (PACK - 1))

    def issue(j, buf, sem):
        pltpu.async_copy(up_hbm.at[idx1p_v.at[j]], u_v.at[buf], sem)
        pltpu.async_copy(cp_hbm.at[idx2p_v.at[j]], c_v.at[buf], sem)

    def drain(j, buf, sem):
        pltpu.make_async_copy(
            up_hbm.at[pl.ds(0, CHUNK), pl.ds(0, VROW)], u_v.at[buf], sem).wait()
        pltpu.make_async_copy(
            cp_hbm.at[pl.ds(0, CHUNK), pl.ds(0, VROW)], c_v.at[buf], sem).wait()

    iota16 = lax.iota(jnp.int32, LANES)

    def compute(j, buf):
        @pl.loop(0, GROUPS)
        def _(g):
            rbase = g * LANES
            par1 = (lax.shift_right_logical(
                idx1_v[j, pl.ds(rbase, LANES)], LOG2_PACK) & 1) * EMB_DIM
            par2 = (lax.shift_right_logical(
                idx2_v[j, pl.ds(rbase, LANES)], LOG2_PACK) & 1) * EMB_DIM
            for i in range(LANES):
                r = rbase + i
                uoff = par1[i]
                coff = par2[i]
                acc = (u_v[buf, r, pl.ds(uoff, LANES)]
                       * c_v[buf, r, pl.ds(coff, LANES)])
                for k in range(1, EMB_DIM // LANES):
                    acc = acc + (u_v[buf, r, pl.ds(uoff + k * LANES, LANES)]
                                 * c_v[buf, r, pl.ds(coff + k * LANES, LANES)])
                plsc.store_scatter(
                    pad_v, [iota16, jnp.full((LANES,), i, jnp.int32)], acc)
            dots = pad_v[0, pl.ds(0, LANES)]
            for l in range(1, LANES):
                dots = dots + pad_v[l, pl.ds(0, LANES)]
            col = pl.ds(rbase, LANES)
            tot = dots + float(EMB_DIM) * bs_v[j, col]
            out_v[pl.ds(j * CHUNK + rbase, LANES)] = 1.0 / (1.0 + jnp.exp(-tot))

    issue(0, 0, sems[0])
    for j in range(NCHUNK):
        if j + 1 < NCHUNK:
            issue(j + 1, (j + 1) % 2, sems[(j + 1) % 2])
        drain(j, j % 2, sems[j % 2])
        compute(j, j % 2)

    pltpu.sync_copy(out_v, out_hbm.at[pl.ds(base, B_PER_W)])


N_U = 1000000
N_C = 100000
PACK = 8192
LOG2_PACK = 13
BLKL = 2 * PACK
LOG2_BLKL = LOG2_PACK + 1
NBLK_U = (N_U + BLKL - 1) // BLKL
NBLK_C = (N_C + BLKL - 1) // BLKL


def _repack_body(t_ref, o_ref):
    o_ref[:, :EMB_DIM] = t_ref[:, :PACK].T
    o_ref[:, EMB_DIM:] = t_ref[:, PACK:].T


def _repack_tc(table_t, nblk):
    return pl.pallas_call(
        _repack_body,
        grid=(nblk,),
        in_specs=[pl.BlockSpec((EMB_DIM, BLKL), lambda j: (0, j))],
        out_specs=pl.BlockSpec((PACK, VROW), lambda j: (j, 0)),
        out_shape=jax.ShapeDtypeStruct((nblk * PACK, VROW), jnp.float32),
        compiler_params=pltpu.CompilerParams(
            dimension_semantics=("parallel",)),
    )(table_t)


def kernel(x1, x2, uemb, cemb, user_bias, creator_bias):
    x1 = x1.astype(jnp.int32)
    x2 = x2.astype(jnp.int32)
    bsum = _bias_sc(x1, x2, user_bias.T.reshape(-1), creator_bias.T.reshape(-1))
    up = _repack_tc(uemb.T, NBLK_U)
    cp = _repack_tc(cemb.T, NBLK_C)
    return _ranker_sc(x1, x2, up, cp, bsum)


# speedup vs baseline: 16.0011x; 1.0341x over previous
import functools

import jax
import jax.numpy as jnp
from jax import lax
from jax.experimental import pallas as pl
from jax.experimental.pallas import tpu as pltpu
from jax.experimental.pallas import tpu_sc as plsc

EMB_DIM = 64
BATCH = 16384

NUM_CORES = 2
NUM_SUBCORES = 16
LANES = 16
NW = NUM_CORES * NUM_SUBCORES
B_PER_W = BATCH // NW
CHUNK = 128
NCHUNK = B_PER_W // CHUNK
GROUPS = CHUNK // LANES
VROW = 2 * EMB_DIM

_mesh = plsc.VectorSubcoreMesh(
    core_axis_name="c", subcore_axis_name="s",
    num_cores=NUM_CORES, num_subcores=NUM_SUBCORES,
)


@functools.partial(
    pl.kernel,
    out_type=jax.ShapeDtypeStruct((BATCH,), jnp.float32),
    mesh=_mesh,
    scratch_types=[
        pltpu.VMEM((NCHUNK, CHUNK), jnp.int32),
        pltpu.VMEM((NCHUNK, CHUNK), jnp.int32),
        pltpu.VMEM((NCHUNK, CHUNK), jnp.float32),
        pltpu.VMEM((NCHUNK, CHUNK), jnp.float32),
        pltpu.SemaphoreType.DMA,
    ],
    compiler_params=pltpu.CompilerParams(
        needs_layout_passes=False, use_tc_tiling_on_sc=False),
)
def _bias_sc(x1_hbm, x2_hbm, ubias_hbm, cbias_hbm, out_hbm,
             idx1_v, idx2_v, b1_v, b2_v, sem):
    wid = lax.axis_index("s") * NUM_CORES + lax.axis_index("c")
    base = wid * B_PER_W

    for j in range(NCHUNK):
        pltpu.sync_copy(x1_hbm.at[pl.ds(base + j * CHUNK, CHUNK)], idx1_v.at[j])
        pltpu.sync_copy(x2_hbm.at[pl.ds(base + j * CHUNK, CHUNK)], idx2_v.at[j])

    copies = []
    for j in range(NCHUNK):
        copies.append(pltpu.async_copy(ubias_hbm.at[idx1_v.at[j]], b1_v.at[j], sem))
        copies.append(pltpu.async_copy(cbias_hbm.at[idx2_v.at[j]], b2_v.at[j], sem))
    for cp in copies:
        cp.wait()

    for j in range(NCHUNK):
        @pl.loop(0, GROUPS)
        def _(g):
            col = pl.ds(g * LANES, LANES)
            b1_v[j, col] = b1_v[j, col] + b2_v[j, col]

        pltpu.sync_copy(b1_v.at[j], out_hbm.at[pl.ds(base + j * CHUNK, CHUNK)])


@functools.partial(
    pl.kernel,
    out_type=jax.ShapeDtypeStruct((BATCH,), jnp.float32),
    mesh=_mesh,
    scratch_types=[
        pltpu.VMEM((NCHUNK, CHUNK), jnp.int32),
        pltpu.VMEM((NCHUNK, CHUNK), jnp.int32),
        pltpu.VMEM((NCHUNK, CHUNK), jnp.int32),
        pltpu.VMEM((NCHUNK, CHUNK), jnp.int32),
        pltpu.VMEM((2, CHUNK, VROW), jnp.float32),
        pltpu.VMEM((2, CHUNK, VROW), jnp.float32),
        pltpu.VMEM((NCHUNK, CHUNK), jnp.float32),
        pltpu.VMEM((B_PER_W,), jnp.float32),
        pltpu.VMEM((LANES, LANES + 1), jnp.float32),
        pltpu.SemaphoreType.DMA,
        pltpu.SemaphoreType.DMA,
    ],
    compiler_params=pltpu.CompilerParams(
        needs_layout_passes=False, use_tc_tiling_on_sc=True),
)
def _ranker_sc(x1_hbm, x2_hbm, up_hbm, cp_hbm, bsum_hbm,
               out_hbm, idx1_v, idx2_v, idx1p_v, idx2p_v, u_v, c_v, bs_v,
               out_v, pad_v, sem0, sem1):
    wid = lax.axis_index("s") * NUM_CORES + lax.axis_index("c")
    base = wid * B_PER_W
    sems = [sem0, sem1]

    for j in range(NCHUNK):
        pltpu.sync_copy(x1_hbm.at[pl.ds(base + j * CHUNK, CHUNK)], idx1_v.at[j])
        pltpu.sync_copy(x2_hbm.at[pl.ds(base + j * CHUNK, CHUNK)], idx2_v.at[j])
        pltpu.sync_copy(bsum_hbm.at[pl.ds(base + j * CHUNK, CHUNK)], bs_v.at[j])

        @pl.loop(0, GROUPS)
        def _(g):
            col = pl.ds(g * LANES, LANES)
            v1 = idx1_v[j, col]
            v2 = idx2_v[j, col]
            idx1p_v[j, col] = lax.shift_left(
                lax.shift_right_logical(v1, LOG2_BLKL), LOG2_PACK) + (
                    v1 & (PACK - 1))
            idx2p_v[j, col] = lax.shift_left(
                lax.shift_right_logical(v2, LOG2_BLKL), LOG2_PACK) + (
                    v2 & (PACK - 1))

    def issue(j, buf, sem):
        pltpu.async_copy(up_hbm.at[idx1p_v.at[j]], u_v.at[buf], sem)
        pltpu.async_copy(cp_hbm.at[idx2p_v.at[j]], c_v.at[buf], sem)

    def drain(j, buf, sem):
        pltpu.make_async_copy(
            up_hbm.at[pl.ds(0, CHUNK), pl.ds(0, VROW)], u_v.at[buf], sem).wait()
        pltpu.make_async_copy(
            cp_hbm.at[pl.ds(0, CHUNK), pl.ds(0, VROW)], c_v.at[buf], sem).wait()

    iota16 = lax.iota(jnp.int32, LANES)

    def compute(j, buf):
        @pl.loop(0, GROUPS)
        def _(g):
            rbase = g * LANES
            par1 = (lax.shift_right_logical(
                idx1_v[j, pl.ds(rbase, LANES)], LOG2_PACK) & 1) * EMB_DIM
            par2 = (lax.shift_right_logical(
                idx2_v[j, pl.ds(rbase, LANES)], LOG2_PACK) & 1) * EMB_DIM
            for i in range(LANES):
                r = rbase + i
                uoff = par1[i]
                coff = par2[i]
                acc = (u_v[buf, r, pl.ds(uoff, LANES)]
                       * c_v[buf, r, pl.ds(coff, LANES)])
                for k in range(1, EMB_DIM // LANES):
                    acc = acc + (u_v[buf, r, pl.ds(uoff + k * LANES, LANES)]
                                 * c_v[buf, r, pl.ds(coff + k * LANES, LANES)])
                plsc.store_scatter(
                    pad_v, [iota16, jnp.full((LANES,), i, jnp.int32)], acc)
            dots = pad_v[0, pl.ds(0, LANES)]
            for l in range(1, LANES):
                dots = dots + pad_v[l, pl.ds(0, LANES)]
            col = pl.ds(rbase, LANES)
            tot = dots + float(EMB_DIM) * bs_v[j, col]
            out_v[pl.ds(j * CHUNK + rbase, LANES)] = 1.0 / (1.0 + jnp.exp(-tot))

    issue(0, 0, sems[0])
    for j in range(NCHUNK):
        if j + 1 < NCHUNK:
            issue(j + 1, (j + 1) % 2, sems[(j + 1) % 2])
        drain(j, j % 2, sems[j % 2])
        compute(j, j % 2)

    pltpu.sync_copy(out_v, out_hbm.at[pl.ds(base, B_PER_W)])


N_U = 1000000
N_C = 100000
PACK = 16384
LOG2_PACK = 14
BLKL = 2 * PACK
LOG2_BLKL = LOG2_PACK + 1
NBLK_U = (N_U + BLKL - 1) // BLKL
NBLK_C = (N_C + BLKL - 1) // BLKL


def _repack_body(t_ref, o_ref):
    o_ref[:, :EMB_DIM] = t_ref[:, :PACK].T
    o_ref[:, EMB_DIM:] = t_ref[:, PACK:].T


def _repack_tc(table_t, nblk):
    return pl.pallas_call(
        _repack_body,
        grid=(nblk,),
        in_specs=[pl.BlockSpec((EMB_DIM, BLKL), lambda j: (0, j))],
        out_specs=pl.BlockSpec((PACK, VROW), lambda j: (j, 0)),
        out_shape=jax.ShapeDtypeStruct((nblk * PACK, VROW), jnp.float32),
        compiler_params=pltpu.CompilerParams(
            dimension_semantics=("parallel",)),
    )(table_t)


def kernel(x1, x2, uemb, cemb, user_bias, creator_bias):
    x1 = x1.astype(jnp.int32)
    x2 = x2.astype(jnp.int32)
    bsum = _bias_sc(x1, x2, user_bias.T.reshape(-1), creator_bias.T.reshape(-1))
    up = _repack_tc(uemb.T, NBLK_U)
    cp = _repack_tc(cemb.T, NBLK_C)
    return _ranker_sc(x1, x2, up, cp, bsum)
